# Initial kernel scaffold; baseline (speedup 1.0000x reference)
import jax
import jax.numpy as jnp
from jax.experimental import pallas as pl


def kernel(x, edge_index, W1, b1, W2, b2, Wfc, bfc):
    raise NotImplementedError("write your pallas kernel here")


# speedup vs baseline: 19.4992x; 19.4992x over previous
import functools

import jax
import jax.numpy as jnp
from jax import lax
from jax.experimental import pallas as pl
from jax.experimental.pallas import tpu as pltpu
from jax.experimental.pallas import tpu_sc as plsc

N = 50000
E = 800000
IN_DIM = 16
HID = 64
OUT_DIM = 1

NPAD = 50176
EPAD = 819200
ECH = EPAD // 128
STRIPE = NPAD // 16

_f32 = jnp.float32
_i32 = jnp.int32


def _deg_body(dst2, zeros, ones, counts, acc, didx, obuf, ssem):
    c = lax.axis_index("c")
    s = lax.axis_index("s")
    wid = c * 16 + s
    pltpu.sync_copy(ones, obuf)
    pltpu.sync_copy(zeros.at[pl.ds(s * STRIPE, STRIPE)],
                    acc.at[pl.ds(s * STRIPE, STRIPE)])
    plsc.subcore_barrier()
    base = wid * (ECH // 32)
    @pl.loop(0, 25)
    def _grp(g):
        pltpu.sync_copy(dst2.at[pl.ds(base + g * 8, 8)], didx)
        descs = [
            pltpu.async_copy(obuf, acc.at[didx.at[b]], ssem, add=True)
            for b in range(8)
        ]
        for d in descs:
            d.wait()
    plsc.subcore_barrier()
    pltpu.sync_copy(acc.at[pl.ds(s * STRIPE, STRIPE)],
                    counts.at[c, pl.ds(s * STRIPE, STRIPE)])


def _make_layer_body(fh, grp):
    ngrp = (ECH // 16) // grp

    def body(src2, dst2, ytab, out, acc, sidx, didx, gbuf, gsem, ssem):
        c = lax.axis_index("c")
        s = lax.axis_index("s")
        pltpu.sync_copy(ytab.at[c, pl.ds(s * STRIPE, STRIPE)],
                        acc.at[pl.ds(s * STRIPE, STRIPE)])
        plsc.subcore_barrier()
        base = s * (ECH // 16)
        @pl.loop(0, ngrp)
        def _grp(g):
            row0 = base + g * grp
            pltpu.sync_copy(src2.at[pl.ds(row0, grp)], sidx)
            pltpu.sync_copy(dst2.at[pl.ds(row0, grp)], didx)
            gds = [
                pltpu.async_copy(ytab.at[c].at[sidx.at[b]], gbuf.at[b], gsem)
                for b in range(grp)
            ]
            for d in gds:
                d.wait()
            sds = [
                pltpu.async_copy(gbuf.at[b], acc.at[didx.at[b]], ssem, add=True)
                for b in range(grp)
            ]
            for d in sds:
                d.wait()
        plsc.subcore_barrier()
        pltpu.sync_copy(acc.at[pl.ds(s * STRIPE, STRIPE)],
                        out.at[c, pl.ds(s * STRIPE, STRIPE)])
    return body


_MESH = plsc.VectorSubcoreMesh(core_axis_name="c", subcore_axis_name="s")


def _sc_degree(dst2, zeros, ones):
    return pl.kernel(
        _deg_body,
        out_type=jax.ShapeDtypeStruct((2, NPAD, 8), _f32),
        mesh=_MESH,
        compiler_params=pltpu.CompilerParams(use_tc_tiling_on_sc=False),
        scratch_types=[
            pltpu.VMEM_SHARED((NPAD, 8), _f32),
            pltpu.VMEM((8, 128), _i32),
            pltpu.VMEM((128, 8), _f32),
            pltpu.SemaphoreType.DMA,
        ],
    )(dst2, zeros, ones)


def _sc_layer(src2, dst2, ytab, fh):
    grp = 4 if fh == 32 else 8
    return pl.kernel(
        _make_layer_body(fh, grp),
        out_type=jax.ShapeDtypeStruct((2, NPAD, fh), _f32),
        mesh=_MESH,
        compiler_params=pltpu.CompilerParams(use_tc_tiling_on_sc=False),
        scratch_types=[
            pltpu.VMEM_SHARED((NPAD, fh), _f32),
            pltpu.VMEM((grp, 128), _i32),
            pltpu.VMEM((grp, 128), _i32),
            pltpu.VMEM((grp, 128, fh), _f32),
            pltpu.SemaphoreType.DMA,
            pltpu.SemaphoreType.DMA,
        ],
    )(src2, dst2, ytab)


_BLK = NPAD // 8


def _tcA_body(x_ref, w1_ref, cnt_ref, y1_ref, dis_ref):
    xw = jnp.dot(x_ref[...], w1_ref[...], preferred_element_type=_f32)
    deg = cnt_ref[0, :, 0:1] + cnt_ref[1, :, 0:1] + 1.0
    dis = lax.rsqrt(deg)
    y = xw * dis
    y1_ref[0] = y[:, :32]
    y1_ref[1] = y[:, 32:]
    dis_ref[...] = dis


def _tcB_body(agg_ref, dis_ref, b1_ref, w2_ref, y2_ref):
    dis = dis_ref[...]
    h_lo = jax.nn.relu(agg_ref[0] * dis + b1_ref[0:32])
    h_hi = jax.nn.relu(agg_ref[1] * dis + b1_ref[32:64])
    hw = (jnp.dot(h_lo, w2_ref[0:32, :], preferred_element_type=_f32)
          + jnp.dot(h_hi, w2_ref[32:64, :], preferred_element_type=_f32))
    y2 = hw * dis
    y2_ref[0] = y2[:, :16]
    y2_ref[1] = y2[:, 16:]


def _tcC_body(agg_ref, dis_ref, b2_ref, wfc_ref, bfc_ref, out_ref):
    i = pl.program_id(0)
    dis = dis_ref[...]
    h_lo = jax.nn.relu(agg_ref[0] * dis + b2_ref[0:16])
    h_hi = jax.nn.relu(agg_ref[1] * dis + b2_ref[16:32])
    row = lax.broadcasted_iota(_i32, (_BLK, 1), 0) + i * _BLK
    mask = row < N
    h_lo = jnp.where(mask, h_lo, 0.0)
    h_hi = jnp.where(mask, h_hi, 0.0)
    s_lo = jnp.sum(h_lo, axis=0)
    s_hi = jnp.sum(h_hi, axis=0)
    part = (jnp.sum(s_lo * wfc_ref[0:16, 0]) + jnp.sum(s_hi * wfc_ref[16:32, 0]))
    part = part * (1.0 / N)

    @pl.when(i == 0)
    def _():
        out_ref[...] = bfc_ref[...] + part

    @pl.when(i != 0)
    def _():
        out_ref[...] = out_ref[...] + part


def _tcA(x_pad, W1, counts):
    return pl.pallas_call(
        _tcA_body,
        grid=(8,),
        in_specs=[
            pl.BlockSpec((_BLK, IN_DIM), lambda i: (i, 0)),
            pl.BlockSpec((IN_DIM, HID), lambda i: (0, 0)),
            pl.BlockSpec((2, _BLK, 8), lambda i: (0, i, 0)),
        ],
        out_specs=[
            pl.BlockSpec((2, _BLK, 32), lambda i: (0, i, 0)),
            pl.BlockSpec((_BLK, 1), lambda i: (i, 0)),
        ],
        out_shape=[
            jax.ShapeDtypeStruct((2, NPAD, 32), _f32),
            jax.ShapeDtypeStruct((NPAD, 1), _f32),
        ],
    )(x_pad, W1, counts)


def _tcB(agg1, dis, b1, W2):
    return pl.pallas_call(
        _tcB_body,
        grid=(8,),
        in_specs=[
            pl.BlockSpec((2, _BLK, 32), lambda i: (0, i, 0)),
            pl.BlockSpec((_BLK, 1), lambda i: (i, 0)),
            pl.BlockSpec((HID,), lambda i: (0,)),
            pl.BlockSpec((HID, HID // 2), lambda i: (0, 0)),
        ],
        out_specs=pl.BlockSpec((2, _BLK, 16), lambda i: (0, i, 0)),
        out_shape=jax.ShapeDtypeStruct((2, NPAD, 16), _f32),
    )(agg1, dis, b1, W2)


def _tcC(agg2, dis, b2, Wfc, bfc):
    return pl.pallas_call(
        _tcC_body,
        grid=(8,),
        in_specs=[
            pl.BlockSpec((2, _BLK, 16), lambda i: (0, i, 0)),
            pl.BlockSpec((_BLK, 1), lambda i: (i, 0)),
            pl.BlockSpec((HID // 2,), lambda i: (0,)),
            pl.BlockSpec((HID // 2, OUT_DIM), lambda i: (0, 0)),
            pl.BlockSpec((1, OUT_DIM), lambda i: (0, 0)),
        ],
        out_specs=pl.BlockSpec((1, OUT_DIM), lambda i: (0, 0)),
        out_shape=jax.ShapeDtypeStruct((1, OUT_DIM), _f32),
    )(agg2, dis, b2, Wfc, bfc.reshape(1, OUT_DIM))


def kernel(x, edge_index, W1, b1, W2, b2, Wfc, bfc):
    x_pad = jnp.pad(x, ((0, NPAD - N), (0, 0)))
    pad = jnp.full((2, EPAD - E), N, dtype=edge_index.dtype)
    ei = jnp.concatenate([edge_index, pad], axis=1)
    src2 = ei[0].reshape(ECH, 128)
    dst2 = ei[1].reshape(ECH, 128)
    zeros = jnp.zeros((NPAD, 8), _f32)
    ones = jnp.ones((128, 8), _f32)

    counts = _sc_degree(dst2, zeros, ones)
    y1, dis = _tcA(x_pad, W1, counts)
    agg1 = _sc_layer(src2, dst2, y1, 32)
    y2 = _tcB(agg1, dis, b1, W2)
    agg2 = _sc_layer(src2, dst2, y2, 16)
    return _tcC(agg2, dis, b2, Wfc, bfc).reshape(OUT_DIM)


# speedup vs baseline: 20.4251x; 1.0475x over previous
import functools

import jax
import jax.numpy as jnp
from jax import lax
from jax.experimental import pallas as pl
from jax.experimental.pallas import tpu as pltpu
from jax.experimental.pallas import tpu_sc as plsc

N = 50000
E = 800000
IN_DIM = 16
HID = 64
OUT_DIM = 1

NPAD = 50176
EPAD = 819200
ECH = EPAD // 128
STRIPE = NPAD // 16

_f32 = jnp.float32
_i32 = jnp.int32


def _deg_body(dst2, zeros, ones, counts, acc, didx, obuf, ssem):
    c = lax.axis_index("c")
    s = lax.axis_index("s")
    wid = c * 16 + s
    pltpu.sync_copy(ones, obuf)
    pltpu.sync_copy(zeros.at[pl.ds(s * STRIPE, STRIPE)],
                    acc.at[pl.ds(s * STRIPE, STRIPE)])
    plsc.subcore_barrier()
    base = wid * (ECH // 32)
    @pl.loop(0, 25)
    def _grp(g):
        pltpu.sync_copy(dst2.at[pl.ds(base + g * 8, 8)], didx)
        descs = [
            pltpu.async_copy(obuf, acc.at[didx.at[b]], ssem, add=True)
            for b in range(8)
        ]
        for d in descs:
            d.wait()
    plsc.subcore_barrier()
    pltpu.sync_copy(acc.at[pl.ds(s * STRIPE, STRIPE)],
                    counts.at[c, pl.ds(s * STRIPE, STRIPE)])


def _make_layer_body(fh, grp):
    ngrp = (ECH // 16) // grp

    def body(src2, dst2, ytab, out, acc, sidx, didx, gbuf, gsem0, gsem1,
             ssem0, ssem1):
        c = lax.axis_index("c")
        s = lax.axis_index("s")
        pltpu.sync_copy(ytab.at[c, pl.ds(s * STRIPE, STRIPE)],
                        acc.at[pl.ds(s * STRIPE, STRIPE)])
        plsc.subcore_barrier()
        base = s * (ECH // 16)
        gsems = (gsem0, gsem1)
        ssems = (ssem0, ssem1)

        def load_idx(p, g):
            row0 = base + g * grp
            pltpu.sync_copy(src2.at[pl.ds(row0, grp)], sidx.at[p])
            pltpu.sync_copy(dst2.at[pl.ds(row0, grp)], didx.at[p])

        def fire_g(p):
            for b in range(grp):
                pltpu.async_copy(ytab.at[c].at[sidx.at[p].at[b]],
                                 gbuf.at[p].at[b], gsems[p])

        def drain_g(p):
            for b in range(grp):
                pltpu.make_async_copy(ytab.at[c, pl.ds(0, 128)],
                                      gbuf.at[p].at[b], gsems[p]).wait()

        def fire_s(p):
            for b in range(grp):
                pltpu.async_copy(gbuf.at[p].at[b], acc.at[didx.at[p].at[b]],
                                 ssems[p], add=True)

        def drain_s(p):
            for b in range(grp):
                pltpu.make_async_copy(gbuf.at[p].at[b], acc.at[pl.ds(0, 128)],
                                      ssems[p]).wait()

        load_idx(0, 0)
        fire_g(0)
        load_idx(1, 1)
        fire_g(1)

        @pl.loop(0, ngrp // 2 - 1)
        def _steady(i):
            g = 2 * i
            drain_g(0)
            fire_s(0)
            drain_g(1)
            fire_s(1)
            drain_s(0)
            load_idx(0, g + 2)
            fire_g(0)
            drain_s(1)
            load_idx(1, g + 3)
            fire_g(1)

        drain_g(0)
        fire_s(0)
        drain_g(1)
        fire_s(1)
        drain_s(0)
        drain_s(1)
        plsc.subcore_barrier()
        pltpu.sync_copy(acc.at[pl.ds(s * STRIPE, STRIPE)],
                        out.at[c, pl.ds(s * STRIPE, STRIPE)])
    return body


_MESH = plsc.VectorSubcoreMesh(core_axis_name="c", subcore_axis_name="s")


def _sc_degree(dst2, zeros, ones):
    return pl.kernel(
        _deg_body,
        out_type=jax.ShapeDtypeStruct((2, NPAD, 8), _f32),
        mesh=_MESH,
        compiler_params=pltpu.CompilerParams(use_tc_tiling_on_sc=False),
        scratch_types=[
            pltpu.VMEM_SHARED((NPAD, 8), _f32),
            pltpu.VMEM((8, 128), _i32),
            pltpu.VMEM((128, 8), _f32),
            pltpu.SemaphoreType.DMA,
        ],
    )(dst2, zeros, ones)


def _sc_layer(src2, dst2, ytab, fh):
    grp = 2 if fh == 32 else 4
    return pl.kernel(
        _make_layer_body(fh, grp),
        out_type=jax.ShapeDtypeStruct((2, NPAD, fh), _f32),
        mesh=_MESH,
        compiler_params=pltpu.CompilerParams(use_tc_tiling_on_sc=False),
        scratch_types=[
            pltpu.VMEM_SHARED((NPAD, fh), _f32),
            pltpu.VMEM((2, grp, 128), _i32),
            pltpu.VMEM((2, grp, 128), _i32),
            pltpu.VMEM((2, grp, 128, fh), _f32),
            pltpu.SemaphoreType.DMA,
            pltpu.SemaphoreType.DMA,
            pltpu.SemaphoreType.DMA,
            pltpu.SemaphoreType.DMA,
        ],
    )(src2, dst2, ytab)


_BLK = NPAD // 8


def _tcA_body(x_ref, w1_ref, cnt_ref, y1_ref, dis_ref):
    xw = jnp.dot(x_ref[...], w1_ref[...], preferred_element_type=_f32)
    deg = cnt_ref[0, :, 0:1] + cnt_ref[1, :, 0:1] + 1.0
    dis = lax.rsqrt(deg)
    y = xw * dis
    y1_ref[0] = y[:, :32]
    y1_ref[1] = y[:, 32:]
    dis_ref[...] = dis


def _tcB_body(agg_ref, dis_ref, b1_ref, w2_ref, y2_ref):
    dis = dis_ref[...]
    h_lo = jax.nn.relu(agg_ref[0] * dis + b1_ref[0:32])
    h_hi = jax.nn.relu(agg_ref[1] * dis + b1_ref[32:64])
    hw = (jnp.dot(h_lo, w2_ref[0:32, :], preferred_element_type=_f32)
          + jnp.dot(h_hi, w2_ref[32:64, :], preferred_element_type=_f32))
    y2 = hw * dis
    y2_ref[0] = y2[:, :16]
    y2_ref[1] = y2[:, 16:]


def _tcC_body(agg_ref, dis_ref, b2_ref, wfc_ref, bfc_ref, out_ref):
    i = pl.program_id(0)
    dis = dis_ref[...]
    h_lo = jax.nn.relu(agg_ref[0] * dis + b2_ref[0:16])
    h_hi = jax.nn.relu(agg_ref[1] * dis + b2_ref[16:32])
    row = lax.broadcasted_iota(_i32, (_BLK, 1), 0) + i * _BLK
    mask = row < N
    h_lo = jnp.where(mask, h_lo, 0.0)
    h_hi = jnp.where(mask, h_hi, 0.0)
    s_lo = jnp.sum(h_lo, axis=0)
    s_hi = jnp.sum(h_hi, axis=0)
    part = (jnp.sum(s_lo * wfc_ref[0:16, 0]) + jnp.sum(s_hi * wfc_ref[16:32, 0]))
    part = part * (1.0 / N)

    @pl.when(i == 0)
    def _():
        out_ref[...] = bfc_ref[...] + part

    @pl.when(i != 0)
    def _():
        out_ref[...] = out_ref[...] + part


def _tcA(x_pad, W1, counts):
    return pl.pallas_call(
        _tcA_body,
        grid=(8,),
        in_specs=[
            pl.BlockSpec((_BLK, IN_DIM), lambda i: (i, 0)),
            pl.BlockSpec((IN_DIM, HID), lambda i: (0, 0)),
            pl.BlockSpec((2, _BLK, 8), lambda i: (0, i, 0)),
        ],
        out_specs=[
            pl.BlockSpec((2, _BLK, 32), lambda i: (0, i, 0)),
            pl.BlockSpec((_BLK, 1), lambda i: (i, 0)),
        ],
        out_shape=[
            jax.ShapeDtypeStruct((2, NPAD, 32), _f32),
            jax.ShapeDtypeStruct((NPAD, 1), _f32),
        ],
    )(x_pad, W1, counts)


def _tcB(agg1, dis, b1, W2):
    return pl.pallas_call(
        _tcB_body,
        grid=(8,),
        in_specs=[
            pl.BlockSpec((2, _BLK, 32), lambda i: (0, i, 0)),
            pl.BlockSpec((_BLK, 1), lambda i: (i, 0)),
            pl.BlockSpec((HID,), lambda i: (0,)),
            pl.BlockSpec((HID, HID // 2), lambda i: (0, 0)),
        ],
        out_specs=pl.BlockSpec((2, _BLK, 16), lambda i: (0, i, 0)),
        out_shape=jax.ShapeDtypeStruct((2, NPAD, 16), _f32),
    )(agg1, dis, b1, W2)


def _tcC(agg2, dis, b2, Wfc, bfc):
    return pl.pallas_call(
        _tcC_body,
        grid=(8,),
        in_specs=[
            pl.BlockSpec((2, _BLK, 16), lambda i: (0, i, 0)),
            pl.BlockSpec((_BLK, 1), lambda i: (i, 0)),
            pl.BlockSpec((HID // 2,), lambda i: (0,)),
            pl.BlockSpec((HID // 2, OUT_DIM), lambda i: (0, 0)),
            pl.BlockSpec((1, OUT_DIM), lambda i: (0, 0)),
        ],
        out_specs=pl.BlockSpec((1, OUT_DIM), lambda i: (0, 0)),
        out_shape=jax.ShapeDtypeStruct((1, OUT_DIM), _f32),
    )(agg2, dis, b2, Wfc, bfc.reshape(1, OUT_DIM))


def kernel(x, edge_index, W1, b1, W2, b2, Wfc, bfc):
    x_pad = jnp.pad(x, ((0, NPAD - N), (0, 0)))
    pad = jnp.full((2, EPAD - E), N, dtype=edge_index.dtype)
    ei = jnp.concatenate([edge_index, pad], axis=1)
    src2 = ei[0].reshape(ECH, 128)
    dst2 = ei[1].reshape(ECH, 128)
    zeros = jnp.zeros((NPAD, 8), _f32)
    ones = jnp.ones((128, 8), _f32)

    counts = _sc_degree(dst2, zeros, ones)
    y1, dis = _tcA(x_pad, W1, counts)
    agg1 = _sc_layer(src2, dst2, y1, 32)
    y2 = _tcB(agg1, dis, b1, W2)
    agg2 = _sc_layer(src2, dst2, y2, 16)
    return _tcC(agg2, dis, b2, Wfc, bfc).reshape(OUT_DIM)


# speedup vs baseline: 23.0963x; 1.1308x over previous
import functools

import jax
import jax.numpy as jnp
from jax import lax
from jax.experimental import pallas as pl
from jax.experimental.pallas import tpu as pltpu
from jax.experimental.pallas import tpu_sc as plsc

N = 50000
E = 800000
IN_DIM = 16
HID = 64
OUT_DIM = 1

NPAD = 50176
EPAD = 819200
ECH = EPAD // 128
STRIPE = NPAD // 16

_f32 = jnp.float32
_i32 = jnp.int32


def _deg_body(dst2, zeros, ones, counts, acc, didx, obuf, ssem):
    c = lax.axis_index("c")
    s = lax.axis_index("s")
    wid = c * 16 + s
    pltpu.sync_copy(ones, obuf)
    base = wid * (ECH // 32)
    pltpu.sync_copy(dst2.at[pl.ds(base, ECH // 32)], didx)
    pltpu.sync_copy(zeros.at[pl.ds(s * STRIPE, STRIPE)],
                    acc.at[pl.ds(s * STRIPE, STRIPE)])
    plsc.subcore_barrier()
    @pl.loop(0, 25)
    def _grp(g):
        descs = [
            pltpu.async_copy(obuf, acc.at[didx.at[g * 8 + b]], ssem, add=True)
            for b in range(8)
        ]
        for d in descs:
            d.wait()
    plsc.subcore_barrier()
    pltpu.sync_copy(acc.at[pl.ds(s * STRIPE, STRIPE)],
                    counts.at[c, pl.ds(s * STRIPE, STRIPE)])


def _make_layer_body(fh, grp, sb_rows):
    ngrp_sb = sb_rows // grp
    nsb = (ECH // 16) // sb_rows

    def body(src2, dst2, ytab, out, acc, sidx, didx, gbuf,
             isem0, isem1, gsem0, gsem1, ssem0, ssem1):
        c = lax.axis_index("c")
        s = lax.axis_index("s")
        pltpu.sync_copy(ytab.at[c, pl.ds(s * STRIPE, STRIPE)],
                        acc.at[pl.ds(s * STRIPE, STRIPE)])
        plsc.subcore_barrier()
        base = s * (ECH // 16)
        isems = (isem0, isem1)
        gsems = (gsem0, gsem1)
        ssems = (ssem0, ssem1)

        def fire_idx(a, sb):
            row0 = base + sb * sb_rows
            pltpu.async_copy(src2.at[pl.ds(row0, sb_rows)], sidx.at[a],
                             isems[a])
            pltpu.async_copy(dst2.at[pl.ds(row0, sb_rows)], didx.at[a],
                             isems[a])

        def wait_idx(a):
            pltpu.make_async_copy(src2.at[pl.ds(0, sb_rows)], sidx.at[a],
                                  isems[a]).wait()
            pltpu.make_async_copy(dst2.at[pl.ds(0, sb_rows)], didx.at[a],
                                  isems[a]).wait()

        def fire_g(a, p, j):
            for b in range(grp):
                pltpu.async_copy(ytab.at[c].at[sidx.at[a].at[j * grp + b]],
                                 gbuf.at[p].at[b], gsems[p])

        def drain_g(p):
            for b in range(grp):
                pltpu.make_async_copy(ytab.at[c, pl.ds(0, 128)],
                                      gbuf.at[p].at[b], gsems[p]).wait()

        def fire_s(a, p, j):
            for b in range(grp):
                pltpu.async_copy(gbuf.at[p].at[b],
                                 acc.at[didx.at[a].at[j * grp + b]],
                                 ssems[p], add=True)

        def drain_s(p):
            for b in range(grp):
                pltpu.make_async_copy(gbuf.at[p].at[b], acc.at[pl.ds(0, 128)],
                                      ssems[p]).wait()

        fire_idx(0, 0)

        @pl.loop(0, nsb // 2)
        def _sb2(k):
            for a in (0, 1):
                sb = 2 * k + a
                wait_idx(a)

                @pl.when(sb < nsb - 1)
                def _():
                    fire_idx(1 - a, sb + 1)

                fire_g(a, 0, 0)
                fire_g(a, 1, 1)
                for j in range(0, ngrp_sb, 2):
                    drain_g(0)
                    fire_s(a, 0, j)
                    drain_g(1)
                    fire_s(a, 1, j + 1)
                    drain_s(0)
                    if j + 2 < ngrp_sb:
                        fire_g(a, 0, j + 2)
                    drain_s(1)
                    if j + 3 < ngrp_sb:
                        fire_g(a, 1, j + 3)

        plsc.subcore_barrier()
        pltpu.sync_copy(acc.at[pl.ds(s * STRIPE, STRIPE)],
                        out.at[c, pl.ds(s * STRIPE, STRIPE)])
    return body


_MESH = plsc.VectorSubcoreMesh(core_axis_name="c", subcore_axis_name="s")


def _sc_degree(dst2, zeros, ones):
    return pl.kernel(
        _deg_body,
        out_type=jax.ShapeDtypeStruct((2, NPAD, 8), _f32),
        mesh=_MESH,
        compiler_params=pltpu.CompilerParams(use_tc_tiling_on_sc=False),
        scratch_types=[
            pltpu.VMEM_SHARED((NPAD, 8), _f32),
            pltpu.VMEM((ECH // 32, 128), _i32),
            pltpu.VMEM((128, 8), _f32),
            pltpu.SemaphoreType.DMA,
        ],
    )(dst2, zeros, ones)


def _sc_layer(src2, dst2, ytab, fh):
    grp = 2 if fh == 32 else 4
    sb_rows = 20 if fh == 32 else 40
    return pl.kernel(
        _make_layer_body(fh, grp, sb_rows),
        out_type=jax.ShapeDtypeStruct((2, NPAD, fh), _f32),
        mesh=_MESH,
        compiler_params=pltpu.CompilerParams(use_tc_tiling_on_sc=False),
        scratch_types=[
            pltpu.VMEM_SHARED((NPAD, fh), _f32),
            pltpu.VMEM((2, sb_rows, 128), _i32),
            pltpu.VMEM((2, sb_rows, 128), _i32),
            pltpu.VMEM((2, grp, 128, fh), _f32),
            pltpu.SemaphoreType.DMA,
            pltpu.SemaphoreType.DMA,
            pltpu.SemaphoreType.DMA,
            pltpu.SemaphoreType.DMA,
            pltpu.SemaphoreType.DMA,
            pltpu.SemaphoreType.DMA,
        ],
    )(src2, dst2, ytab)


_BLK = NPAD // 8


def _tcA_body(x_ref, w1_ref, cnt_ref, y1_ref, dis_ref):
    xw = jnp.dot(x_ref[...], w1_ref[...], preferred_element_type=_f32)
    deg = cnt_ref[0, :, 0:1] + cnt_ref[1, :, 0:1] + 1.0
    dis = lax.rsqrt(deg)
    y = xw * dis
    y1_ref[0] = y[:, :32]
    y1_ref[1] = y[:, 32:]
    dis_ref[...] = dis


def _tcB_body(agg_ref, dis_ref, b1_ref, w2_ref, y2_ref):
    dis = dis_ref[...]
    h_lo = jax.nn.relu(agg_ref[0] * dis + b1_ref[0:32])
    h_hi = jax.nn.relu(agg_ref[1] * dis + b1_ref[32:64])
    hw = (jnp.dot(h_lo, w2_ref[0:32, :], preferred_element_type=_f32)
          + jnp.dot(h_hi, w2_ref[32:64, :], preferred_element_type=_f32))
    y2 = hw * dis
    y2_ref[0] = y2[:, :16]
    y2_ref[1] = y2[:, 16:]


def _tcC_body(agg_ref, dis_ref, b2_ref, wfc_ref, bfc_ref, out_ref):
    i = pl.program_id(0)
    dis = dis_ref[...]
    h_lo = jax.nn.relu(agg_ref[0] * dis + b2_ref[0:16])
    h_hi = jax.nn.relu(agg_ref[1] * dis + b2_ref[16:32])
    row = lax.broadcasted_iota(_i32, (_BLK, 1), 0) + i * _BLK
    mask = row < N
    h_lo = jnp.where(mask, h_lo, 0.0)
    h_hi = jnp.where(mask, h_hi, 0.0)
    s_lo = jnp.sum(h_lo, axis=0)
    s_hi = jnp.sum(h_hi, axis=0)
    part = (jnp.sum(s_lo * wfc_ref[0:16, 0]) + jnp.sum(s_hi * wfc_ref[16:32, 0]))
    part = part * (1.0 / N)

    @pl.when(i == 0)
    def _():
        out_ref[...] = bfc_ref[...] + part

    @pl.when(i != 0)
    def _():
        out_ref[...] = out_ref[...] + part


def _tcA(x_pad, W1, counts):
    return pl.pallas_call(
        _tcA_body,
        grid=(8,),
        in_specs=[
            pl.BlockSpec((_BLK, IN_DIM), lambda i: (i, 0)),
            pl.BlockSpec((IN_DIM, HID), lambda i: (0, 0)),
            pl.BlockSpec((2, _BLK, 8), lambda i: (0, i, 0)),
        ],
        out_specs=[
            pl.BlockSpec((2, _BLK, 32), lambda i: (0, i, 0)),
            pl.BlockSpec((_BLK, 1), lambda i: (i, 0)),
        ],
        out_shape=[
            jax.ShapeDtypeStruct((2, NPAD, 32), _f32),
            jax.ShapeDtypeStruct((NPAD, 1), _f32),
        ],
    )(x_pad, W1, counts)


def _tcB(agg1, dis, b1, W2):
    return pl.pallas_call(
        _tcB_body,
        grid=(8,),
        in_specs=[
            pl.BlockSpec((2, _BLK, 32), lambda i: (0, i, 0)),
            pl.BlockSpec((_BLK, 1), lambda i: (i, 0)),
            pl.BlockSpec((HID,), lambda i: (0,)),
            pl.BlockSpec((HID, HID // 2), lambda i: (0, 0)),
        ],
        out_specs=pl.BlockSpec((2, _BLK, 16), lambda i: (0, i, 0)),
        out_shape=jax.ShapeDtypeStruct((2, NPAD, 16), _f32),
    )(agg1, dis, b1, W2)


def _tcC(agg2, dis, b2, Wfc, bfc):
    return pl.pallas_call(
        _tcC_body,
        grid=(8,),
        in_specs=[
            pl.BlockSpec((2, _BLK, 16), lambda i: (0, i, 0)),
            pl.BlockSpec((_BLK, 1), lambda i: (i, 0)),
            pl.BlockSpec((HID // 2,), lambda i: (0,)),
            pl.BlockSpec((HID // 2, OUT_DIM), lambda i: (0, 0)),
            pl.BlockSpec((1, OUT_DIM), lambda i: (0, 0)),
        ],
        out_specs=pl.BlockSpec((1, OUT_DIM), lambda i: (0, 0)),
        out_shape=jax.ShapeDtypeStruct((1, OUT_DIM), _f32),
    )(agg2, dis, b2, Wfc, bfc.reshape(1, OUT_DIM))


def kernel(x, edge_index, W1, b1, W2, b2, Wfc, bfc):
    x_pad = jnp.pad(x, ((0, NPAD - N), (0, 0)))
    pad = jnp.full((2, EPAD - E), N, dtype=edge_index.dtype)
    ei = jnp.concatenate([edge_index, pad], axis=1)
    src2 = ei[0].reshape(ECH, 128)
    dst2 = ei[1].reshape(ECH, 128)
    zeros = jnp.zeros((NPAD, 8), _f32)
    ones = jnp.ones((128, 8), _f32)

    counts = _sc_degree(dst2, zeros, ones)
    y1, dis = _tcA(x_pad, W1, counts)
    agg1 = _sc_layer(src2, dst2, y1, 32)
    y2 = _tcB(agg1, dis, b1, W2)
    agg2 = _sc_layer(src2, dst2, y2, 16)
    return _tcC(agg2, dis, b2, Wfc, bfc).reshape(OUT_DIM)


# speedup vs baseline: 32.8078x; 1.4205x over previous
import jax
import jax.numpy as jnp
from jax import lax
from jax.experimental import pallas as pl
from jax.experimental.pallas import tpu as pltpu
from jax.experimental.pallas import tpu_sc as plsc

N = 50000
E = 800000
IN_DIM = 16
HID = 64
OUT_DIM = 1

NPAD = 50176
EPAD = 819200
ECH = EPAD // 128
STRIPE = NPAD // 16

_f32 = jnp.float32
_i32 = jnp.int32


def _deg_body(dst2, zeros, ones, counts, acc, didx, obuf, ssem):
    c = lax.axis_index("c")
    s = lax.axis_index("s")
    wid = c * 16 + s
    pltpu.sync_copy(ones, obuf)
    base = wid * (ECH // 32)
    pltpu.sync_copy(dst2.at[pl.ds(base, ECH // 32)], didx)
    pltpu.sync_copy(zeros.at[pl.ds(s * STRIPE, STRIPE)],
                    acc.at[pl.ds(s * STRIPE, STRIPE)])
    plsc.subcore_barrier()
    @pl.loop(0, 25)
    def _grp(g):
        descs = [
            pltpu.async_copy(obuf, acc.at[didx.at[g * 8 + b]], ssem, add=True)
            for b in range(8)
        ]
        for d in descs:
            d.wait()
    plsc.subcore_barrier()
    pltpu.sync_copy(acc.at[pl.ds(s * STRIPE, STRIPE)],
                    counts.at[c, pl.ds(s * STRIPE, STRIPE)])


def _make_layer_body(npass):
    grp = 2
    sb_rows = 20
    ngrp_sb = sb_rows // grp
    nsb = (ECH // 16) // sb_rows

    def body(src2, dst2, ytab, out, tab, acc, sidx, didx, gbuf,
             isem0, isem1, gsem0, gsem1, ssem0, ssem1):
        c = lax.axis_index("c")
        s = lax.axis_index("s")
        base = s * (ECH // 16)
        isems = (isem0, isem1)
        gsems = (gsem0, gsem1)
        ssems = (ssem0, ssem1)

        def fire_idx(a, sb):
            row0 = base + sb * sb_rows
            pltpu.async_copy(src2.at[pl.ds(row0, sb_rows)], sidx.at[a],
                             isems[a])
            pltpu.async_copy(dst2.at[pl.ds(row0, sb_rows)], didx.at[a],
                             isems[a])

        def wait_idx(a):
            pltpu.make_async_copy(src2.at[pl.ds(0, sb_rows)], sidx.at[a],
                                  isems[a]).wait()
            pltpu.make_async_copy(dst2.at[pl.ds(0, sb_rows)], didx.at[a],
                                  isems[a]).wait()

        def fire_g(a, p, j):
            for b in range(grp):
                pltpu.async_copy(tab.at[sidx.at[a].at[j * grp + b]],
                                 gbuf.at[p].at[b], gsems[p])

        def drain_g(p):
            for b in range(grp):
                pltpu.make_async_copy(ytab.at[0, pl.ds(0, 128)],
                                      gbuf.at[p].at[b], gsems[p]).wait()

        def fire_s(a, p, j):
            for b in range(grp):
                pltpu.async_copy(gbuf.at[p].at[b],
                                 acc.at[didx.at[a].at[j * grp + b]],
                                 ssems[p], add=True)

        def drain_s(p):
            for b in range(grp):
                pltpu.make_async_copy(gbuf.at[p].at[b], acc.at[pl.ds(0, 128)],
                                      ssems[p]).wait()

        for ph in range(npass):
            q = c * npass + ph
            pltpu.sync_copy(ytab.at[q].at[pl.ds(s * STRIPE, STRIPE)],
                            tab.at[pl.ds(s * STRIPE, STRIPE)])
            pltpu.sync_copy(ytab.at[q].at[pl.ds(s * STRIPE, STRIPE)],
                            acc.at[pl.ds(s * STRIPE, STRIPE)])
            plsc.subcore_barrier()

            fire_idx(0, 0)

            @pl.loop(0, nsb // 2)
            def _sb2(k):
                for a in (0, 1):
                    sb = 2 * k + a
                    wait_idx(a)

                    @pl.when(sb < nsb - 1)
                    def _():
                        fire_idx(1 - a, sb + 1)

                    fire_g(a, 0, 0)
                    fire_g(a, 1, 1)
                    for j in range(0, ngrp_sb, 2):
                        drain_g(0)
                        fire_s(a, 0, j)
                        drain_g(1)
                        fire_s(a, 1, j + 1)
                        drain_s(0)
                        if j + 2 < ngrp_sb:
                            fire_g(a, 0, j + 2)
                        drain_s(1)
                        if j + 3 < ngrp_sb:
                            fire_g(a, 1, j + 3)

            plsc.subcore_barrier()
            pltpu.sync_copy(acc.at[pl.ds(s * STRIPE, STRIPE)],
                            out.at[q].at[pl.ds(s * STRIPE, STRIPE)])
    return body


_MESH = plsc.VectorSubcoreMesh(core_axis_name="c", subcore_axis_name="s")


def _sc_degree(dst2, zeros, ones):
    return pl.kernel(
        _deg_body,
        out_type=jax.ShapeDtypeStruct((2, NPAD, 8), _f32),
        mesh=_MESH,
        compiler_params=pltpu.CompilerParams(use_tc_tiling_on_sc=False),
        scratch_types=[
            pltpu.VMEM_SHARED((NPAD, 8), _f32),
            pltpu.VMEM((ECH // 32, 128), _i32),
            pltpu.VMEM((128, 8), _f32),
            pltpu.SemaphoreType.DMA,
        ],
    )(dst2, zeros, ones)


def _sc_layer(src2, dst2, ytab, npass):
    return pl.kernel(
        _make_layer_body(npass),
        out_type=jax.ShapeDtypeStruct((2 * npass, NPAD, 16), _f32),
        mesh=_MESH,
        compiler_params=pltpu.CompilerParams(use_tc_tiling_on_sc=False),
        scratch_types=[
            pltpu.VMEM_SHARED((NPAD, 16), _f32),
            pltpu.VMEM_SHARED((NPAD, 16), _f32),
            pltpu.VMEM((2, 20, 128), _i32),
            pltpu.VMEM((2, 20, 128), _i32),
            pltpu.VMEM((2, 2, 128, 16), _f32),
            pltpu.SemaphoreType.DMA,
            pltpu.SemaphoreType.DMA,
            pltpu.SemaphoreType.DMA,
            pltpu.SemaphoreType.DMA,
            pltpu.SemaphoreType.DMA,
            pltpu.SemaphoreType.DMA,
        ],
    )(src2, dst2, ytab)


_BLK = NPAD // 8


def _tcA_body(x_ref, w1_ref, cnt_ref, y1_ref, dis_ref):
    xw = jnp.dot(x_ref[...], w1_ref[...], preferred_element_type=_f32)
    deg = cnt_ref[0, :, 0:1] + cnt_ref[1, :, 0:1] + 1.0
    dis = lax.rsqrt(deg)
    y = xw * dis
    for q in range(4):
        y1_ref[q] = y[:, 16 * q:16 * (q + 1)]
    dis_ref[...] = dis


def _tcB_body(agg_ref, dis_ref, b1_ref, w2_ref, y2_ref):
    dis = dis_ref[...]
    hw = jnp.zeros((_BLK, HID // 2), _f32)
    for q in range(4):
        h_q = jax.nn.relu(agg_ref[q] * dis + b1_ref[16 * q:16 * (q + 1)])
        hw = hw + jnp.dot(h_q, w2_ref[16 * q:16 * (q + 1), :],
                          preferred_element_type=_f32)
    y2 = hw * dis
    y2_ref[0] = y2[:, :16]
    y2_ref[1] = y2[:, 16:]


def _tcC_body(agg_ref, dis_ref, b2_ref, wfc_ref, bfc_ref, out_ref):
    i = pl.program_id(0)
    dis = dis_ref[...]
    h_lo = jax.nn.relu(agg_ref[0] * dis + b2_ref[0:16])
    h_hi = jax.nn.relu(agg_ref[1] * dis + b2_ref[16:32])
    row = lax.broadcasted_iota(_i32, (_BLK, 1), 0) + i * _BLK
    mask = row < N
    h_lo = jnp.where(mask, h_lo, 0.0)
    h_hi = jnp.where(mask, h_hi, 0.0)
    s_lo = jnp.sum(h_lo, axis=0)
    s_hi = jnp.sum(h_hi, axis=0)
    part = (jnp.sum(s_lo * wfc_ref[0:16, 0]) + jnp.sum(s_hi * wfc_ref[16:32, 0]))
    part = part * (1.0 / N)

    @pl.when(i == 0)
    def _():
        out_ref[...] = bfc_ref[...] + part

    @pl.when(i != 0)
    def _():
        out_ref[...] = out_ref[...] + part


def _tcA(x_pad, W1, counts):
    return pl.pallas_call(
        _tcA_body,
        grid=(8,),
        in_specs=[
            pl.BlockSpec((_BLK, IN_DIM), lambda i: (i, 0)),
            pl.BlockSpec((IN_DIM, HID), lambda i: (0, 0)),
            pl.BlockSpec((2, _BLK, 8), lambda i: (0, i, 0)),
        ],
        out_specs=[
            pl.BlockSpec((4, _BLK, 16), lambda i: (0, i, 0)),
            pl.BlockSpec((_BLK, 1), lambda i: (i, 0)),
        ],
        out_shape=[
            jax.ShapeDtypeStruct((4, NPAD, 16), _f32),
            jax.ShapeDtypeStruct((NPAD, 1), _f32),
        ],
    )(x_pad, W1, counts)


def _tcB(agg1, dis, b1, W2):
    return pl.pallas_call(
        _tcB_body,
        grid=(8,),
        in_specs=[
            pl.BlockSpec((4, _BLK, 16), lambda i: (0, i, 0)),
            pl.BlockSpec((_BLK, 1), lambda i: (i, 0)),
            pl.BlockSpec((HID,), lambda i: (0,)),
            pl.BlockSpec((HID, HID // 2), lambda i: (0, 0)),
        ],
        out_specs=pl.BlockSpec((2, _BLK, 16), lambda i: (0, i, 0)),
        out_shape=jax.ShapeDtypeStruct((2, NPAD, 16), _f32),
    )(agg1, dis, b1, W2)


def _tcC(agg2, dis, b2, Wfc, bfc):
    return pl.pallas_call(
        _tcC_body,
        grid=(8,),
        in_specs=[
            pl.BlockSpec((2, _BLK, 16), lambda i: (0, i, 0)),
            pl.BlockSpec((_BLK, 1), lambda i: (i, 0)),
            pl.BlockSpec((HID // 2,), lambda i: (0,)),
            pl.BlockSpec((HID // 2, OUT_DIM), lambda i: (0, 0)),
            pl.BlockSpec((1, OUT_DIM), lambda i: (0, 0)),
        ],
        out_specs=pl.BlockSpec((1, OUT_DIM), lambda i: (0, 0)),
        out_shape=jax.ShapeDtypeStruct((1, OUT_DIM), _f32),
    )(agg2, dis, b2, Wfc, bfc.reshape(1, OUT_DIM))


def kernel(x, edge_index, W1, b1, W2, b2, Wfc, bfc):
    x_pad = jnp.pad(x, ((0, NPAD - N), (0, 0)))
    pad = jnp.full((2, EPAD - E), N, dtype=edge_index.dtype)
    ei = jnp.concatenate([edge_index, pad], axis=1)
    src2 = ei[0].reshape(ECH, 128)
    dst2 = ei[1].reshape(ECH, 128)
    zeros = jnp.zeros((NPAD, 8), _f32)
    ones = jnp.ones((128, 8), _f32)

    counts = _sc_degree(dst2, zeros, ones)
    y1, dis = _tcA(x_pad, W1, counts)
    agg1 = _sc_layer(src2, dst2, y1, 2)
    y2 = _tcB(agg1, dis, b1, W2)
    agg2 = _sc_layer(src2, dst2, y2, 1)
    return _tcC(agg2, dis, b2, Wfc, bfc).reshape(OUT_DIM)
